# baseline (speedup 1.0000x reference)
import functools

import jax
import jax.numpy as jnp
from jax import lax
from jax.experimental import pallas as pl
from jax.experimental.pallas import tpu as pltpu
from jax.experimental.pallas import tpu_sc as plsc

_ROWS_PER_WORKER = 8


@functools.lru_cache(maxsize=None)
def _make_sc_gather(num_rows, d_model, max_len):
    info = plsc.get_sparse_core_info()
    num_cores = info.num_cores
    mesh = plsc.VectorSubcoreMesh(core_axis_name="c", subcore_axis_name="s")
    active = num_rows // _ROWS_PER_WORKER

    @functools.partial(
        pl.kernel,
        mesh=mesh,
        out_type=jax.ShapeDtypeStruct((num_rows, d_model), jnp.float32),
        scratch_types=[
            pltpu.VMEM((_ROWS_PER_WORKER,), jnp.int32),
            pltpu.VMEM((_ROWS_PER_WORKER, d_model), jnp.float32),
            pltpu.SemaphoreType.DMA,
        ],
    )
    def gather(pe_hbm, idx_hbm, out_hbm, idx_v, rows_v, sem):
        wid = lax.axis_index("s") * num_cores + lax.axis_index("c")

        @pl.when(wid < active)
        def _():
            base = wid * _ROWS_PER_WORKER
            pltpu.sync_copy(idx_hbm.at[pl.ds(base, _ROWS_PER_WORKER)], idx_v)
            pltpu.async_copy(pe_hbm.at[idx_v], rows_v, sem).wait()
            pltpu.sync_copy(rows_v, out_hbm.at[pl.ds(base, _ROWS_PER_WORKER)])

    return gather


def _add_body(x_ref, t_ref, o_ref):
    o_ref[...] = x_ref[...] + t_ref[...]


def kernel(x, batch_positions, pe):
    B, T, d_model, H, W = x.shape
    BT = B * T
    HW = H * W
    x3 = x.transpose(0, 1, 3, 4, 2).reshape(BT, HW, d_model)
    pos = batch_positions.reshape(BT)

    table = _make_sc_gather(BT, d_model, pe.shape[0])(pe, pos)
    t3 = table.reshape(BT, 1, d_model)

    G = 24
    out = pl.pallas_call(
        _add_body,
        grid=(BT // G,),
        in_specs=[
            pl.BlockSpec((G, HW, d_model), lambda i: (i, 0, 0)),
            pl.BlockSpec((G, 1, d_model), lambda i: (i, 0, 0)),
        ],
        out_specs=pl.BlockSpec((G, HW, d_model), lambda i: (i, 0, 0)),
        out_shape=jax.ShapeDtypeStruct((BT, HW, d_model), jnp.float32),
    )(x3, t3)
    return out.reshape(B, T, H, W, d_model).transpose(0, 1, 4, 2, 3)


# speedup vs baseline: 1.0176x; 1.0176x over previous
import functools

import jax
import jax.numpy as jnp
from jax import lax
from jax.experimental import pallas as pl
from jax.experimental.pallas import tpu as pltpu
from jax.experimental.pallas import tpu_sc as plsc

_ROWS_PER_WORKER = 8


@functools.lru_cache(maxsize=None)
def _make_sc_gather(num_rows, d_model, max_len):
    info = plsc.get_sparse_core_info()
    num_cores = info.num_cores
    mesh = plsc.VectorSubcoreMesh(core_axis_name="c", subcore_axis_name="s", num_cores=1)
    active = num_rows // _ROWS_PER_WORKER

    @functools.partial(
        pl.kernel,
        mesh=mesh,
        out_type=jax.ShapeDtypeStruct((num_rows, d_model), jnp.float32),
        scratch_types=[
            pltpu.VMEM((_ROWS_PER_WORKER,), jnp.int32),
            pltpu.VMEM((_ROWS_PER_WORKER, d_model), jnp.float32),
            pltpu.SemaphoreType.DMA,
        ],
    )
    def gather(pe_hbm, idx_hbm, out_hbm, idx_v, rows_v, sem):
        wid = lax.axis_index("s") * num_cores + lax.axis_index("c")

        @pl.when(wid < active)
        def _():
            base = wid * _ROWS_PER_WORKER
            pltpu.sync_copy(idx_hbm.at[pl.ds(base, _ROWS_PER_WORKER)], idx_v)
            pltpu.async_copy(pe_hbm.at[idx_v], rows_v, sem).wait()
            pltpu.sync_copy(rows_v, out_hbm.at[pl.ds(base, _ROWS_PER_WORKER)])

    return gather


def _add_body(x_ref, t_ref, o_ref):
    o_ref[...] = x_ref[...] + t_ref[...]


def kernel(x, batch_positions, pe):
    B, T, d_model, H, W = x.shape
    BT = B * T
    HW = H * W
    x3 = x.transpose(0, 1, 3, 4, 2).reshape(BT, HW, d_model)
    pos = batch_positions.reshape(BT)

    table = _make_sc_gather(BT, d_model, pe.shape[0])(pe, pos)
    t3 = table.reshape(BT, 1, d_model)

    G = 16
    out = pl.pallas_call(
        _add_body,
        grid=(BT // G,),
        in_specs=[
            pl.BlockSpec((G, HW, d_model), lambda i: (i, 0, 0)),
            pl.BlockSpec((G, 1, d_model), lambda i: (i, 0, 0)),
        ],
        out_specs=pl.BlockSpec((G, HW, d_model), lambda i: (i, 0, 0)),
        out_shape=jax.ShapeDtypeStruct((BT, HW, d_model), jnp.float32),
    )(x3, t3)
    return out.reshape(B, T, H, W, d_model).transpose(0, 1, 4, 2, 3)
